# baseline (speedup 1.0000x reference)
import jax
import jax.numpy as jnp
from jax import lax
from jax.experimental import pallas as pl
from jax.experimental.pallas import tpu as pltpu
from jax.experimental.pallas import tpu_sc as plsc

N = 10000
NP = 10240
NF = 128
RD = 16
ROW = 160
B = 32
NTILES = 16
EN = 160000
EH = 50000
ENP = 160768
EHP = 50176
BLKN = ENP // (NTILES * B)
BLKH = EHP // (NTILES * B)
ALPHA = 0.2
NB = 4
NBLK = NP // NB


def _elu(v):
    return jnp.where(v > 0.0, v, jnp.exp(jnp.minimum(v, 0.0)) - 1.0)


def _dotT(x, w):
    return lax.dot_general(x, w, (((1,), (1,)), ((), ())),
                           preferred_element_type=jnp.float32)


def _k1a_body(x_ref, a0_ref, a1_ref, a20_ref, a21_ref, u0_ref, u1_ref, ns_ref):
    x = x_ref[...]
    pus, pvs = [], []
    for a_ref, a2_ref, u_ref in ((a0_ref, a20_ref, u0_ref),
                                 (a1_ref, a21_ref, u1_ref)):
        a = a_ref[...]
        a2 = a2_ref[...]
        u = _dotT(x, a[:, :NF])
        u_ref[...] = u
        pus.append(u @ a2[0])
        pvs.append(x @ (a2[0] @ a[:, NF:2 * NF]))
    z = jnp.zeros_like(pus[0])
    ns_ref[...] = jnp.stack([pus[0], pus[1], pvs[0], pvs[1], z, z, z, z])


def _k1b_body(ee_ref, a0_ref, a1_ref, a20_ref, a21_ref, pwn_ref):
    ee = ee_ref[...]
    c0 = a20_ref[...][0] @ a0_ref[...][:, 2 * NF:]
    c1 = a21_ref[...][0] @ a1_ref[...][:, 2 * NF:]
    pwn_ref[...] = jnp.stack([ee @ c0, ee @ c1])


def _k1c_body(rel_ref, w_ref, a0_ref, a1_ref, ao_ref, a20_ref, a21_ref,
              a2o_ref, or1_ref, relpw_ref, war2_ref, misc_ref):
    rel = rel_ref[...]
    w = w_ref[...]
    ao = ao_ref[...]
    D = 2 * NF
    or1 = rel @ w
    or1_ref[...] = or1
    c0 = a20_ref[...][0] @ a0_ref[...][:, 2 * NF:]
    c1 = a21_ref[...][0] @ a1_ref[...][:, 2 * NF:]
    q = a2o_ref[...][0] @ ao[:, 2 * D:]
    wq = w @ q
    pad = jnp.zeros((12,), jnp.float32)
    relpw_ref[...] = jnp.stack([
        jnp.concatenate([rel @ c0, pad]),
        jnp.concatenate([rel @ c1, pad]),
        jnp.concatenate([rel @ wq, pad]),
        jnp.zeros((512,), jnp.float32)])
    war2_ref[...] = _dotT(w, ao[:, 2 * D:])
    wpv2 = a2o_ref[...][0] @ ao[:, D:2 * D]
    z = jnp.zeros((2 * NF,), jnp.float32)
    misc_ref[...] = jnp.stack([wpv2, z, z, z, z, z, z, z])


def _k2_body(acc0_ref, acc1_ref, u0_ref, u1_ref, b0_ref, b1_ref, ad2t_ref,
             a2o_ref, misc_ref, h0_ref, h1_ref, u2_ref, ns2_ref):
    ad2t = ad2t_ref[...]
    a2o = a2o_ref[...]
    wpv2 = misc_ref[...][0]
    hs = []
    for acc_ref, u_ref, b_ref, h_ref in ((acc0_ref, u0_ref, b0_ref, h0_ref),
                                         (acc1_ref, u1_ref, b1_ref, h1_ref)):
        acc = acc_ref[...]
        den = acc[:, 144]
        c = jnp.where(den == 0.0, 1e-12, den)
        rest = acc @ b_ref[...]
        h = _elu((u_ref[...] * den[:, None] + rest) / c[:, None])
        h_ref[...] = h
        hs.append(h)
    u2 = hs[0] @ ad2t[:NF] + hs[1] @ ad2t[NF:]
    u2_ref[...] = u2
    pu2 = u2 @ a2o[0]
    pv2 = hs[0] @ wpv2[:NF] + hs[1] @ wpv2[NF:]
    z = jnp.zeros_like(pu2)
    ns2_ref[...] = jnp.stack([pu2, pv2, z, z, z, z, z, z])


def _k3_body(acca_ref, accb_ref, u2_ref, b2a_ref, b2b_ref, xf_ref):
    acca = acca_ref[...]
    accb = accb_ref[...]
    den = acca[:, 144]
    c = jnp.where(den == 0.0, 1e-12, den)
    rest = acca @ b2a_ref[...] + accb @ b2b_ref[...]
    xf_ref[...] = _elu((u2_ref[...] * den[:, None] + rest) / c[:, None])


def _full(shape):
    return pl.BlockSpec(shape, lambda i: (0, 0))


_k1a = pl.pallas_call(
    _k1a_body, grid=(NB,),
    in_specs=[pl.BlockSpec((NBLK, NF), lambda i: (i, 0)),
              _full((NF, 2 * NF + RD)), _full((NF, 2 * NF + RD)),
              _full((1, NF)), _full((1, NF))],
    out_specs=[pl.BlockSpec((NBLK, NF), lambda i: (i, 0)),
               pl.BlockSpec((NBLK, NF), lambda i: (i, 0)),
               pl.BlockSpec((8, NBLK), lambda i: (0, i))],
    out_shape=[jax.ShapeDtypeStruct((NP, NF), jnp.float32),
               jax.ShapeDtypeStruct((NP, NF), jnp.float32),
               jax.ShapeDtypeStruct((8, NP), jnp.float32)])

_EB = ENP // 8
_k1b = pl.pallas_call(
    _k1b_body, grid=(8,),
    in_specs=[pl.BlockSpec((_EB, RD), lambda i: (i, 0)),
              _full((NF, 2 * NF + RD)), _full((NF, 2 * NF + RD)),
              _full((1, NF)), _full((1, NF))],
    out_specs=pl.BlockSpec((2, _EB), lambda i: (0, i)),
    out_shape=jax.ShapeDtypeStruct((2, ENP), jnp.float32))

_k1c = pl.pallas_call(
    _k1c_body, grid=(1,),
    in_specs=[_full((500, RD)), _full((RD, 2 * NF)),
              _full((NF, 2 * NF + RD)), _full((NF, 2 * NF + RD)),
              _full((2 * NF, 6 * NF)), _full((1, NF)), _full((1, NF)),
              _full((1, 2 * NF))],
    out_specs=[_full((500, 2 * NF)), _full((4, 512)), _full((RD, 2 * NF)),
               _full((8, 2 * NF))],
    out_shape=[jax.ShapeDtypeStruct((500, 2 * NF), jnp.float32),
               jax.ShapeDtypeStruct((4, 512), jnp.float32),
               jax.ShapeDtypeStruct((RD, 2 * NF), jnp.float32),
               jax.ShapeDtypeStruct((8, 2 * NF), jnp.float32)])

_k2 = pl.pallas_call(
    _k2_body, grid=(NB,),
    in_specs=[pl.BlockSpec((NBLK, ROW), lambda i: (i, 0)),
              pl.BlockSpec((NBLK, ROW), lambda i: (i, 0)),
              pl.BlockSpec((NBLK, NF), lambda i: (i, 0)),
              pl.BlockSpec((NBLK, NF), lambda i: (i, 0)),
              _full((ROW, NF)), _full((ROW, NF)), _full((2 * NF, 2 * NF)),
              _full((1, 2 * NF)), _full((8, 2 * NF))],
    out_specs=[pl.BlockSpec((NBLK, NF), lambda i: (i, 0)),
               pl.BlockSpec((NBLK, NF), lambda i: (i, 0)),
               pl.BlockSpec((NBLK, 2 * NF), lambda i: (i, 0)),
               pl.BlockSpec((8, NBLK), lambda i: (0, i))],
    out_shape=[jax.ShapeDtypeStruct((NP, NF), jnp.float32),
               jax.ShapeDtypeStruct((NP, NF), jnp.float32),
               jax.ShapeDtypeStruct((NP, 2 * NF), jnp.float32),
               jax.ShapeDtypeStruct((8, NP), jnp.float32)])

_k3 = pl.pallas_call(
    _k3_body, grid=(NB,),
    in_specs=[pl.BlockSpec((NBLK, ROW), lambda i: (i, 0)),
              pl.BlockSpec((NBLK, ROW), lambda i: (i, 0)),
              pl.BlockSpec((NBLK, 2 * NF), lambda i: (i, 0)),
              _full((ROW, 2 * NF)), _full((ROW, 2 * NF))],
    out_specs=pl.BlockSpec((NBLK, 2 * NF), lambda i: (i, 0)),
    out_shape=jax.ShapeDtypeStruct((NP, 2 * NF), jnp.float32))


_KEYS = ("hv", "sv", "svo", "ivv", "tix", "dsc",
         "pug2", "eer", "xrows", "orows")


def _set_types():
    return [
        pltpu.VMEM((4, B), jnp.int32),
        pltpu.VMEM((B,), jnp.int32),
        pltpu.VMEM((B,), jnp.int32),
        pltpu.VMEM((2 * B,), jnp.int32),
        pltpu.VMEM((2 * B,), jnp.int32),
        pltpu.VMEM((B,), jnp.int32),
        pltpu.VMEM((2 * B,), jnp.float32),
        pltpu.VMEM((2 * B, RD), jnp.float32),
        pltpu.VMEM((B, NF), jnp.float32),
        pltpu.VMEM((B, ROW), jnp.float32),
    ]


def _make_sc_edge_pass(phase_b):
    mesh = plsc.VectorSubcoreMesh(core_axis_name="c", subcore_axis_name="s")
    nset = len(_KEYS)
    scratch = _set_types() + _set_types() + [
        pltpu.VMEM((B,), jnp.float32),
        pltpu.VMEM((512,), jnp.float32),
        pltpu.SemaphoreType.DMA,
        pltpu.SemaphoreType.DMA,
        pltpu.SemaphoreType.DMA,
        pltpu.SemaphoreType.DMA,
        pltpu.SemaphoreType.DMA,
        pltpu.VMEM_SHARED((NP, ROW), jnp.float32),
    ]

    def body(*refs):
        if phase_b:
            (x_hbm, hdrn, hdrh, rel_hbm, puv_t, rpw_t, out_hbm) = refs[:7]
            nin = 7
        else:
            (x_hbm, hdrn, een, hdrh, rel_hbm, puv_t, rpw_t,
             out_hbm) = refs[:8]
            nin = 8
        sets = [dict(zip(_KEYS, refs[nin:nin + nset])),
                dict(zip(_KEYS, refs[nin + nset:nin + 2 * nset]))]
        (ebuf, rpw_vm, sem_l, sem_g0, sem_g1, sem_s0, sem_s1,
         acc) = refs[nin + 2 * nset:]
        sem_g = (sem_g0, sem_g1)
        sem_s = (sem_s0, sem_s1)
        cid = lax.axis_index("c")
        sid = lax.axis_index("s")
        node_off = cid * NP
        lane = lax.broadcasted_iota(jnp.int32, (16,), 0)
        if phase_b:
            pltpu.sync_copy(rpw_t, rpw_vm)
        else:
            pltpu.sync_copy(rpw_t.at[pl.ds(cid * 512, 512)], rpw_vm)

        z0 = sets[0]["orows"]
        zero = jnp.zeros((16,), jnp.float32)

        def zrow(i, carry):
            for st in sets:
                for j in range(ROW // 16):
                    st["orows"][i, pl.ds(j * 16, 16)] = zero
            return carry

        lax.fori_loop(0, B, zrow, 0)
        r0 = sid * (NP // NTILES)
        for z in range((NP // NTILES) // B):
            pltpu.sync_copy(z0, acc.at[pl.ds(r0 + z * B, B)])
        plsc.subcore_barrier()

        zi = jnp.zeros((16,), jnp.int32)
        for st, sem in ((sets[0], sem_s0), (sets[1], sem_s1)):
            for j in range(B // 16):
                st["dsc"][pl.ds(j * 16, 16)] = zi
            pltpu.async_copy(st["orows"], acc.at[st["dsc"]], sem, add=True)

        def hdr_tab(kind):
            return hdrh if kind else hdrn

        def fire_lin(kind, blk, s):
            pltpu.async_copy(hdr_tab(kind).at[blk], s["hv"], sem_l)
            if kind == 0 and not phase_b:
                base = blk * B
                pltpu.async_copy(een.at[pl.ds(base, B)],
                                 s["eer"].at[pl.ds(0, B)], sem_l)

        def wait_lin(kind, s):
            pltpu.make_async_copy(hdr_tab(kind).at[0], s["hv"], sem_l).wait()
            if kind == 0 and not phase_b:
                pltpu.make_async_copy(een.at[pl.ds(0, B)],
                                      s["eer"].at[pl.ds(0, B)], sem_l).wait()

        def unpack(kind, s):
            obn = jnp.broadcast_to(node_off, (16,)).astype(jnp.int32)
            hv = s["hv"]
            for j in range(B // 16):
                sl = pl.ds(j * 16, 16)
                slB = pl.ds(B + j * 16, 16)
                d16 = hv[0, sl]
                s16 = hv[1, sl]
                if phase_b:
                    s["ivv"][sl] = d16
                    s["ivv"][slB] = s16 + NP
                    s["svo"][sl] = s16 + obn
                else:
                    s["ivv"][sl] = d16 + obn
                    s["ivv"][slB] = s16 + obn + 2 * NP
                    s["sv"][sl] = s16
                if kind:
                    s["tix"][sl] = hv[2, sl]
                    s["tix"][slB] = hv[3, sl]
                elif phase_b:
                    s["sv"][sl] = hv[2, sl]

        def gath_list(kind, s):
            xi = s["svo"] if phase_b else s["sv"]
            if kind == 0:
                if phase_b:
                    return [(rel_hbm.at[s["sv"]], s["eer"].at[pl.ds(0, B)]),
                            (puv_t.at[s["ivv"]], s["pug2"]),
                            (x_hbm.at[xi], s["xrows"])]
                return [(puv_t.at[s["ivv"]], s["pug2"]),
                        (x_hbm.at[xi], s["xrows"])]
            return [(rel_hbm.at[s["tix"]], s["eer"]),
                    (puv_t.at[s["ivv"]], s["pug2"]),
                    (x_hbm.at[xi], s["xrows"])]

        def fire_gath(kind, p):
            for src, dst in gath_list(kind, sets[p]):
                pltpu.async_copy(src, dst, sem_g[p])

        def wait_gath(kind, p):
            for src, dst in gath_list(kind, sets[p]):
                pltpu.make_async_copy(src, dst, sem_g[p]).wait()

        def compute_e(kind, s):
            for j in range(B // 16):
                sl = pl.ds(j * 16, 16)
                slB = pl.ds(B + j * 16, 16)
                p = s["pug2"][sl] + s["pug2"][slB]
                if kind == 0:
                    if phase_b:
                        p = p + plsc.load_gather(rpw_vm, [s["sv"][sl]])
                    else:
                        p = p + plsc.bitcast(s["hv"][2 + cid, sl],
                                             jnp.float32)
                else:
                    p = p + plsc.load_gather(rpw_vm, [s["tix"][sl]])
                    p = p + plsc.load_gather(rpw_vm, [s["tix"][slB]])
                ebuf[sl] = jnp.exp(jnp.minimum(-p, -ALPHA * p))

        def drain_scatter(p):
            s = sets[p]
            pltpu.make_async_copy(s["orows"], acc.at[s["dsc"]],
                                  sem_s[p]).wait()

        def scale_fire_scatter(kind, p):
            s = sets[p]
            orows, xrows, eer = s["orows"], s["xrows"], s["eer"]

            def sblk(j, carry):
                ev16 = ebuf[pl.ds(j * 16, 16)]
                for k in range(16):
                    i = j * 16 + k
                    ev = jnp.broadcast_to(ev16[k], (16,))
                    for c in range(NF // 16):
                        sl = pl.ds(c * 16, 16)
                        orows[i, sl] = xrows[i, sl] * ev
                    ee_row = eer[i, :]
                    if kind:
                        ee_row = ee_row + eer[B + i, :]
                    orows[i, pl.ds(NF, 16)] = ee_row * ev
                    orows[i, pl.ds(NF + 16, 16)] = jnp.where(lane == 0, ev,
                                                             0.0)
                return carry

            lax.fori_loop(0, B // 16, sblk, 0)
            for j in range(B // 16):
                sl = pl.ds(j * 16, 16)
                s["dsc"][sl] = s["hv"][0, sl]
            pltpu.async_copy(orows, acc.at[s["dsc"]], sem_s[p], add=True)

        def run_kind(kind, nblk):
            blk0 = sid * nblk
            fire_lin(kind, blk0, sets[0])
            wait_lin(kind, sets[0])
            unpack(kind, sets[0])
            fire_gath(kind, 0)
            fire_lin(kind, blk0 + 1, sets[1])
            npair = nblk // 2

            def half(k, p, b_next2):
                q = 1 - p
                wait_lin(kind, sets[q])
                unpack(kind, sets[q])
                fire_gath(kind, q)
                wait_gath(kind, p)
                compute_e(kind, sets[p])
                drain_scatter(p)
                scale_fire_scatter(kind, p)
                fire_lin(kind, b_next2, sets[p])

            def pair(k, carry):
                b0 = blk0 + 2 * k
                half(k, 0, b0 + 2)
                half(k, 1, b0 + 3)
                return carry

            lax.fori_loop(0, npair - 1, pair, 0)
            wait_lin(kind, sets[1])
            unpack(kind, sets[1])
            fire_gath(kind, 1)
            wait_gath(kind, 0)
            compute_e(kind, sets[0])
            drain_scatter(0)
            scale_fire_scatter(kind, 0)
            wait_gath(kind, 1)
            compute_e(kind, sets[1])
            drain_scatter(1)
            scale_fire_scatter(kind, 1)

        run_kind(0, BLKN)
        run_kind(1, BLKH)
        drain_scatter(0)
        drain_scatter(1)
        plsc.subcore_barrier()
        pltpu.sync_copy(acc.at[pl.ds(r0, NP // NTILES)],
                        out_hbm.at[cid, pl.ds(r0, NP // NTILES)])

    return pl.kernel(body,
                     out_type=jax.ShapeDtypeStruct((2, NP, ROW), jnp.float32),
                     mesh=mesh, scratch_types=scratch,
                     compiler_params=pltpu.CompilerParams(
                         needs_layout_passes=False,
                         use_tc_tiling_on_sc=False))


_sc_phase_a = _make_sc_edge_pass(phase_b=False)
_sc_phase_b = _make_sc_edge_pass(phase_b=True)


def _pad_i(a, L, fill):
    a = a.astype(jnp.int32)
    return jnp.concatenate([a, jnp.full((L - a.shape[0],), fill, jnp.int32)])


def kernel(Corpus_, batch_inputs, entity_embeddings, relation_embed,
           edge_list, edge_type, edge_embed, edge_list_nhop, edge_type_nhop,
           a0, a2_0, a1, a2_1, W, a_out, a2_out):
    f32 = jnp.float32
    x = entity_embeddings.astype(f32)
    xpad = jnp.concatenate([x, jnp.zeros((NP - N, NF), f32)], axis=0)
    dn = _pad_i(edge_list[0], ENP, N)
    sn = _pad_i(edge_list[1], ENP, 0)
    tn = _pad_i(edge_type, ENP, 0)
    dh = _pad_i(edge_list_nhop[0], EHP, N)
    sh = _pad_i(edge_list_nhop[1], EHP, 0)
    t0h = _pad_i(edge_type_nhop[:, 0], EHP, 0)
    t1h = _pad_i(edge_type_nhop[:, 1], EHP, 0)
    eenp = jnp.concatenate([edge_embed.astype(f32),
                            jnp.zeros((ENP - EN, RD), f32)], axis=0)
    rel = relation_embed.astype(f32)

    u0, u1, ns = _k1a(xpad, a0, a1, a2_0, a2_1)
    pwn = _k1b(eenp, a0, a1, a2_0, a2_1)
    or1, relpw, war2, misc = _k1c(rel, W, a0, a1, a_out, a2_0, a2_1, a2_out)

    pwb = lax.bitcast_convert_type(pwn, jnp.int32)
    hdrn_a = jnp.stack([dn, sn, pwb[0], pwb[1]]).reshape(
        4, ENP // B, B).transpose(1, 0, 2)
    hdrn_b = jnp.stack([dn, sn, tn, jnp.zeros_like(dn)]).reshape(
        4, ENP // B, B).transpose(1, 0, 2)
    hdrh = jnp.stack([dh, sh, t0h, t1h]).reshape(
        4, EHP // B, B).transpose(1, 0, 2)

    puv_cat = jnp.concatenate([ns[0], ns[1], ns[2], ns[3]])
    rpw_cat = jnp.concatenate([relpw[0], relpw[1]])
    acc_a = _sc_phase_a(xpad, hdrn_a, eenp, hdrh, rel, puv_cat, rpw_cat)

    b0 = jnp.concatenate([a0[:, NF:2 * NF + RD],
                          jnp.zeros((NF, ROW - NF - RD), f32)], axis=1).T
    b1 = jnp.concatenate([a1[:, NF:2 * NF + RD],
                          jnp.zeros((NF, ROW - NF - RD), f32)], axis=1).T
    ad2t = a_out[:, :2 * NF].T
    h0, h1, u2, ns2 = _k2(acc_a[0], acc_a[1], u0, u1, b0, b1, ad2t, a2_out,
                          misc)

    x2cat = jnp.concatenate([h0, h1], axis=0)
    puv2 = jnp.concatenate([ns2[0], ns2[1]])
    acc_b = _sc_phase_b(x2cat, hdrn_b, hdrh, rel, puv2, relpw[2])

    D = 2 * NF
    b2a = jnp.concatenate([a_out[:, D:D + NF].T, war2,
                           jnp.zeros((16, D), f32)], axis=0)
    b2b = jnp.concatenate([a_out[:, D + NF:2 * D].T,
                           jnp.zeros((32, D), f32)], axis=0)
    xf = _k3(acc_b[0], acc_b[1], u2, b2a, b2b)
    return xf[:N], or1


# speedup vs baseline: 1.0108x; 1.0108x over previous
import jax
import jax.numpy as jnp
from jax import lax
from jax.experimental import pallas as pl
from jax.experimental.pallas import tpu as pltpu
from jax.experimental.pallas import tpu_sc as plsc

N = 10000
NP = 10240
NF = 128
RD = 16
ROW = 160
B = 32
NTILES = 16
EN = 160000
EH = 50000
ENP = 160768
EHP = 50176
BLKN = ENP // (NTILES * B)
BLKH = EHP // (NTILES * B)
ALPHA = 0.2
NB = 4
NBLK = NP // NB


def _elu(v):
    return jnp.where(v > 0.0, v, jnp.exp(jnp.minimum(v, 0.0)) - 1.0)


def _dotT(x, w):
    return lax.dot_general(x, w, (((1,), (1,)), ((), ())),
                           preferred_element_type=jnp.float32)


def _k1a_body(x_ref, a0_ref, a1_ref, a20_ref, a21_ref, u0_ref, u1_ref, ns_ref):
    x = x_ref[...]
    pus, pvs = [], []
    for a_ref, a2_ref, u_ref in ((a0_ref, a20_ref, u0_ref),
                                 (a1_ref, a21_ref, u1_ref)):
        a = a_ref[...]
        a2 = a2_ref[...]
        u = _dotT(x, a[:, :NF])
        u_ref[...] = u
        pus.append(u @ a2[0])
        pvs.append(x @ (a2[0] @ a[:, NF:2 * NF]))
    z = jnp.zeros_like(pus[0])
    ns_ref[...] = jnp.stack([pus[0], pus[1], pvs[0], pvs[1], z, z, z, z])


def _k1b_body(ee_ref, a0_ref, a1_ref, a20_ref, a21_ref, pwn_ref):
    ee = ee_ref[...]
    c0 = a20_ref[...][0] @ a0_ref[...][:, 2 * NF:]
    c1 = a21_ref[...][0] @ a1_ref[...][:, 2 * NF:]
    pwn_ref[...] = jnp.stack([ee @ c0, ee @ c1])


def _k1c_body(rel_ref, w_ref, a0_ref, a1_ref, ao_ref, a20_ref, a21_ref,
              a2o_ref, or1_ref, relpw_ref, war2_ref, misc_ref):
    rel = rel_ref[...]
    w = w_ref[...]
    ao = ao_ref[...]
    D = 2 * NF
    or1 = rel @ w
    or1_ref[...] = or1
    c0 = a20_ref[...][0] @ a0_ref[...][:, 2 * NF:]
    c1 = a21_ref[...][0] @ a1_ref[...][:, 2 * NF:]
    q = a2o_ref[...][0] @ ao[:, 2 * D:]
    wq = w @ q
    pad = jnp.zeros((12,), jnp.float32)
    relpw_ref[...] = jnp.stack([
        jnp.concatenate([rel @ c0, pad]),
        jnp.concatenate([rel @ c1, pad]),
        jnp.concatenate([rel @ wq, pad]),
        jnp.zeros((512,), jnp.float32)])
    war2_ref[...] = _dotT(w, ao[:, 2 * D:])
    wpv2 = a2o_ref[...][0] @ ao[:, D:2 * D]
    z = jnp.zeros((2 * NF,), jnp.float32)
    misc_ref[...] = jnp.stack([wpv2, z, z, z, z, z, z, z])


def _k2_body(acc0_ref, acc1_ref, u0_ref, u1_ref, b0_ref, b1_ref, ad2t_ref,
             a2o_ref, misc_ref, h0_ref, h1_ref, u2_ref, ns2_ref):
    ad2t = ad2t_ref[...]
    a2o = a2o_ref[...]
    wpv2 = misc_ref[...][0]
    hs = []
    for acc_ref, u_ref, b_ref, h_ref in ((acc0_ref, u0_ref, b0_ref, h0_ref),
                                         (acc1_ref, u1_ref, b1_ref, h1_ref)):
        acc = acc_ref[...]
        den = acc[:, 144]
        c = jnp.where(den == 0.0, 1e-12, den)
        rest = acc @ b_ref[...]
        h = _elu((u_ref[...] * den[:, None] + rest) / c[:, None])
        h_ref[...] = h
        hs.append(h)
    u2 = hs[0] @ ad2t[:NF] + hs[1] @ ad2t[NF:]
    u2_ref[...] = u2
    pu2 = u2 @ a2o[0]
    pv2 = hs[0] @ wpv2[:NF] + hs[1] @ wpv2[NF:]
    z = jnp.zeros_like(pu2)
    ns2_ref[...] = jnp.stack([pu2, pv2, z, z, z, z, z, z])


def _k3_body(acca_ref, accb_ref, u2_ref, b2a_ref, b2b_ref, xf_ref):
    acca = acca_ref[...]
    accb = accb_ref[...]
    den = acca[:, 144]
    c = jnp.where(den == 0.0, 1e-12, den)
    rest = acca @ b2a_ref[...] + accb @ b2b_ref[...]
    xf_ref[...] = _elu((u2_ref[...] * den[:, None] + rest) / c[:, None])


def _full(shape):
    return pl.BlockSpec(shape, lambda i: (0, 0))


_k1a = pl.pallas_call(
    _k1a_body, grid=(NB,),
    in_specs=[pl.BlockSpec((NBLK, NF), lambda i: (i, 0)),
              _full((NF, 2 * NF + RD)), _full((NF, 2 * NF + RD)),
              _full((1, NF)), _full((1, NF))],
    out_specs=[pl.BlockSpec((NBLK, NF), lambda i: (i, 0)),
               pl.BlockSpec((NBLK, NF), lambda i: (i, 0)),
               pl.BlockSpec((8, NBLK), lambda i: (0, i))],
    out_shape=[jax.ShapeDtypeStruct((NP, NF), jnp.float32),
               jax.ShapeDtypeStruct((NP, NF), jnp.float32),
               jax.ShapeDtypeStruct((8, NP), jnp.float32)])

_EB = ENP // 8
_k1b = pl.pallas_call(
    _k1b_body, grid=(8,),
    in_specs=[pl.BlockSpec((_EB, RD), lambda i: (i, 0)),
              _full((NF, 2 * NF + RD)), _full((NF, 2 * NF + RD)),
              _full((1, NF)), _full((1, NF))],
    out_specs=pl.BlockSpec((2, _EB), lambda i: (0, i)),
    out_shape=jax.ShapeDtypeStruct((2, ENP), jnp.float32))

_k1c = pl.pallas_call(
    _k1c_body, grid=(1,),
    in_specs=[_full((500, RD)), _full((RD, 2 * NF)),
              _full((NF, 2 * NF + RD)), _full((NF, 2 * NF + RD)),
              _full((2 * NF, 6 * NF)), _full((1, NF)), _full((1, NF)),
              _full((1, 2 * NF))],
    out_specs=[_full((500, 2 * NF)), _full((4, 512)), _full((RD, 2 * NF)),
               _full((8, 2 * NF))],
    out_shape=[jax.ShapeDtypeStruct((500, 2 * NF), jnp.float32),
               jax.ShapeDtypeStruct((4, 512), jnp.float32),
               jax.ShapeDtypeStruct((RD, 2 * NF), jnp.float32),
               jax.ShapeDtypeStruct((8, 2 * NF), jnp.float32)])

_k2 = pl.pallas_call(
    _k2_body, grid=(NB,),
    in_specs=[pl.BlockSpec((NBLK, ROW), lambda i: (i, 0)),
              pl.BlockSpec((NBLK, ROW), lambda i: (i, 0)),
              pl.BlockSpec((NBLK, NF), lambda i: (i, 0)),
              pl.BlockSpec((NBLK, NF), lambda i: (i, 0)),
              _full((ROW, NF)), _full((ROW, NF)), _full((2 * NF, 2 * NF)),
              _full((1, 2 * NF)), _full((8, 2 * NF))],
    out_specs=[pl.BlockSpec((NBLK, NF), lambda i: (i, 0)),
               pl.BlockSpec((NBLK, NF), lambda i: (i, 0)),
               pl.BlockSpec((NBLK, 2 * NF), lambda i: (i, 0)),
               pl.BlockSpec((8, NBLK), lambda i: (0, i))],
    out_shape=[jax.ShapeDtypeStruct((NP, NF), jnp.float32),
               jax.ShapeDtypeStruct((NP, NF), jnp.float32),
               jax.ShapeDtypeStruct((NP, 2 * NF), jnp.float32),
               jax.ShapeDtypeStruct((8, NP), jnp.float32)])

_k3 = pl.pallas_call(
    _k3_body, grid=(NB,),
    in_specs=[pl.BlockSpec((NBLK, ROW), lambda i: (i, 0)),
              pl.BlockSpec((NBLK, ROW), lambda i: (i, 0)),
              pl.BlockSpec((NBLK, 2 * NF), lambda i: (i, 0)),
              _full((ROW, 2 * NF)), _full((ROW, 2 * NF))],
    out_specs=pl.BlockSpec((NBLK, 2 * NF), lambda i: (i, 0)),
    out_shape=jax.ShapeDtypeStruct((NP, 2 * NF), jnp.float32))


_KEYS = ("hv", "dv", "sv", "dvo", "svo", "t0v", "t1v", "dsc",
         "pwv", "pug", "pvg", "eev", "r1v", "xrows", "orows")


def _set_types():
    return [
        pltpu.VMEM((4, B), jnp.int32),
        pltpu.VMEM((B,), jnp.int32),
        pltpu.VMEM((B,), jnp.int32),
        pltpu.VMEM((B,), jnp.int32),
        pltpu.VMEM((B,), jnp.int32),
        pltpu.VMEM((B,), jnp.int32),
        pltpu.VMEM((B,), jnp.int32),
        pltpu.VMEM((B,), jnp.int32),
        pltpu.VMEM((B,), jnp.float32),
        pltpu.VMEM((B,), jnp.float32),
        pltpu.VMEM((B,), jnp.float32),
        pltpu.VMEM((B, RD), jnp.float32),
        pltpu.VMEM((B, RD), jnp.float32),
        pltpu.VMEM((B, NF), jnp.float32),
        pltpu.VMEM((B, ROW), jnp.float32),
    ]


def _make_sc_edge_pass(phase_b):
    mesh = plsc.VectorSubcoreMesh(core_axis_name="c", subcore_axis_name="s")
    nset = len(_KEYS)
    scratch = _set_types() + _set_types() + [
        pltpu.VMEM((B,), jnp.float32),
        pltpu.VMEM((512,), jnp.float32),
        pltpu.SemaphoreType.DMA,
        pltpu.SemaphoreType.DMA,
        pltpu.SemaphoreType.DMA,
        pltpu.SemaphoreType.DMA,
        pltpu.SemaphoreType.DMA,
        pltpu.VMEM_SHARED((NP, ROW), jnp.float32),
    ]

    def body(*refs):
        if phase_b:
            (x_hbm, hdrn, hdrh, rel_hbm, pu_t, pv_t, rpw_t, out_hbm) = refs[:8]
            nin = 8
        else:
            (x_hbm, hdrn, pwn, een, hdrh, rel_hbm, pu_t, pv_t, rpw_t,
             out_hbm) = refs[:10]
            nin = 10
        sets = [dict(zip(_KEYS, refs[nin:nin + nset])),
                dict(zip(_KEYS, refs[nin + nset:nin + 2 * nset]))]
        (ebuf, rpw_vm, sem_l, sem_g0, sem_g1, sem_s0, sem_s1,
         acc) = refs[nin + 2 * nset:]
        sem_g = (sem_g0, sem_g1)
        sem_s = (sem_s0, sem_s1)
        cid = lax.axis_index("c")
        sid = lax.axis_index("s")
        node_off = cid * NP
        lane = lax.broadcasted_iota(jnp.int32, (16,), 0)
        if phase_b:
            pltpu.sync_copy(rpw_t, rpw_vm)
        else:
            pltpu.sync_copy(rpw_t.at[pl.ds(cid * 512, 512)], rpw_vm)

        z0 = sets[0]["orows"]
        zero = jnp.zeros((16,), jnp.float32)

        def zrow(i, carry):
            for st in sets:
                for j in range(ROW // 16):
                    st["orows"][i, pl.ds(j * 16, 16)] = zero
            return carry

        lax.fori_loop(0, B, zrow, 0)
        r0 = sid * (NP // NTILES)
        zcps = [pltpu.async_copy(z0, acc.at[pl.ds(r0 + z * B, B)], sem_l)
                for z in range((NP // NTILES) // B)]
        for cp in zcps:
            cp.wait()
        plsc.subcore_barrier()

        zi = jnp.zeros((16,), jnp.int32)
        for st, sem in ((sets[0], sem_s0), (sets[1], sem_s1)):
            for j in range(B // 16):
                st["dsc"][pl.ds(j * 16, 16)] = zi
            pltpu.async_copy(st["orows"], acc.at[st["dsc"]], sem, add=True)

        def hdr_tab(kind):
            return hdrh if kind else hdrn

        def fire_lin(kind, blk, s):
            pltpu.async_copy(hdr_tab(kind).at[blk], s["hv"], sem_l)
            if kind == 0 and not phase_b:
                base = blk * B
                pltpu.async_copy(pwn.at[cid, pl.ds(base, B)], s["pwv"], sem_l)
                pltpu.async_copy(een.at[pl.ds(base, B)], s["eev"], sem_l)

        def wait_lin(kind, s):
            pltpu.make_async_copy(hdr_tab(kind).at[0], s["hv"], sem_l).wait()
            if kind == 0 and not phase_b:
                pltpu.make_async_copy(pwn.at[0, pl.ds(0, B)], s["pwv"],
                                      sem_l).wait()
                pltpu.make_async_copy(een.at[pl.ds(0, B)], s["eev"],
                                      sem_l).wait()

        def unpack(kind, s):
            rows = 4 if kind else (3 if phase_b else 2)
            obn = jnp.broadcast_to(node_off, (16,)).astype(jnp.int32)
            hv = s["hv"]
            for j in range(B // 16):
                sl = pl.ds(j * 16, 16)
                d16 = hv[0, sl]
                s16 = hv[1, sl]
                s["dv"][sl] = d16
                s["sv"][sl] = s16
                s["svo"][sl] = s16 + obn
                if not phase_b:
                    s["dvo"][sl] = d16 + obn
                if rows >= 3:
                    s["t0v"][sl] = hv[2, sl]
                if rows >= 4:
                    s["t1v"][sl] = hv[3, sl]

        def gath_list(kind, s):
            if kind == 0:
                if phase_b:
                    return [(rel_hbm.at[s["t0v"]], s["eev"]),
                            (pu_t.at[s["dv"]], s["pug"]),
                            (pv_t.at[s["sv"]], s["pvg"]),
                            (x_hbm.at[s["svo"]], s["xrows"])]
                return [(pu_t.at[s["dvo"]], s["pug"]),
                        (pv_t.at[s["svo"]], s["pvg"]),
                        (x_hbm.at[s["sv"]], s["xrows"])]
            return [(rel_hbm.at[s["t0v"]], s["eev"]),
                    (rel_hbm.at[s["t1v"]], s["r1v"]),
                    (pu_t.at[s["dv"] if phase_b else s["dvo"]], s["pug"]),
                    (pv_t.at[s["sv"] if phase_b else s["svo"]], s["pvg"]),
                    (x_hbm.at[s["svo"] if phase_b else s["sv"]], s["xrows"])]

        def fire_gath(kind, p):
            for src, dst in gath_list(kind, sets[p]):
                pltpu.async_copy(src, dst, sem_g[p])

        def wait_gath(kind, p):
            for src, dst in gath_list(kind, sets[p]):
                pltpu.make_async_copy(src, dst, sem_g[p]).wait()

        def compute_e(kind, s):
            for j in range(B // 16):
                sl = pl.ds(j * 16, 16)
                p = s["pug"][sl] + s["pvg"][sl]
                if kind == 0:
                    if phase_b:
                        p = p + plsc.load_gather(rpw_vm, [s["t0v"][sl]])
                    else:
                        p = p + s["pwv"][sl]
                else:
                    p = p + plsc.load_gather(rpw_vm, [s["t0v"][sl]])
                    p = p + plsc.load_gather(rpw_vm, [s["t1v"][sl]])
                ebuf[sl] = jnp.exp(jnp.minimum(-p, -ALPHA * p))

        def drain_scatter(p):
            s = sets[p]
            pltpu.make_async_copy(s["orows"], acc.at[s["dsc"]],
                                  sem_s[p]).wait()

        def scale_fire_scatter(kind, p):
            s = sets[p]
            orows, xrows, eev, r1v = (s["orows"], s["xrows"], s["eev"],
                                      s["r1v"])

            def sblk(j, carry):
                ev16 = ebuf[pl.ds(j * 16, 16)]
                for k in range(16):
                    i = j * 16 + k
                    ev = jnp.broadcast_to(ev16[k], (16,))
                    for c in range(NF // 16):
                        sl = pl.ds(c * 16, 16)
                        orows[i, sl] = xrows[i, sl] * ev
                    ee_row = eev[i, :]
                    if kind:
                        ee_row = ee_row + r1v[i, :]
                    orows[i, pl.ds(NF, 16)] = ee_row * ev
                    orows[i, pl.ds(NF + 16, 16)] = jnp.where(lane == 0, ev,
                                                             0.0)
                return carry

            lax.fori_loop(0, B // 16, sblk, 0)
            for j in range(B // 16):
                sl = pl.ds(j * 16, 16)
                s["dsc"][sl] = s["dv"][sl]
            pltpu.async_copy(orows, acc.at[s["dsc"]], sem_s[p], add=True)

        def run_kind(kind, nblk):
            blk0 = sid * nblk
            fire_lin(kind, blk0, sets[0])
            wait_lin(kind, sets[0])
            unpack(kind, sets[0])
            fire_gath(kind, 0)
            fire_lin(kind, blk0 + 1, sets[1])
            npair = nblk // 2

            def half(k, p, b_next2):
                q = 1 - p
                wait_lin(kind, sets[q])
                unpack(kind, sets[q])
                fire_gath(kind, q)
                wait_gath(kind, p)
                compute_e(kind, sets[p])
                drain_scatter(p)
                scale_fire_scatter(kind, p)
                fire_lin(kind, b_next2, sets[p])

            def pair(k, carry):
                b0 = blk0 + 2 * k
                half(k, 0, b0 + 2)
                half(k, 1, b0 + 3)
                return carry

            lax.fori_loop(0, npair - 1, pair, 0)
            wait_lin(kind, sets[1])
            unpack(kind, sets[1])
            fire_gath(kind, 1)
            wait_gath(kind, 0)
            compute_e(kind, sets[0])
            drain_scatter(0)
            scale_fire_scatter(kind, 0)
            wait_gath(kind, 1)
            compute_e(kind, sets[1])
            drain_scatter(1)
            scale_fire_scatter(kind, 1)

        run_kind(0, BLKN)
        run_kind(1, BLKH)
        drain_scatter(0)
        drain_scatter(1)
        plsc.subcore_barrier()
        pltpu.sync_copy(acc.at[pl.ds(r0, NP // NTILES)],
                        out_hbm.at[cid, pl.ds(r0, NP // NTILES)])

    return pl.kernel(body,
                     out_type=jax.ShapeDtypeStruct((2, NP, ROW), jnp.float32),
                     mesh=mesh, scratch_types=scratch,
                     compiler_params=pltpu.CompilerParams(
                         needs_layout_passes=False,
                         use_tc_tiling_on_sc=False))


_sc_phase_a = _make_sc_edge_pass(phase_b=False)
_sc_phase_b = _make_sc_edge_pass(phase_b=True)


def _pad_i(a, L, fill):
    a = a.astype(jnp.int32)
    return jnp.concatenate([a, jnp.full((L - a.shape[0],), fill, jnp.int32)])


def kernel(Corpus_, batch_inputs, entity_embeddings, relation_embed,
           edge_list, edge_type, edge_embed, edge_list_nhop, edge_type_nhop,
           a0, a2_0, a1, a2_1, W, a_out, a2_out):
    f32 = jnp.float32
    x = entity_embeddings.astype(f32)
    xpad = jnp.concatenate([x, jnp.zeros((NP - N, NF), f32)], axis=0)
    dn = _pad_i(edge_list[0], ENP, N)
    sn = _pad_i(edge_list[1], ENP, 0)
    tn = _pad_i(edge_type, ENP, 0)
    dh = _pad_i(edge_list_nhop[0], EHP, N)
    sh = _pad_i(edge_list_nhop[1], EHP, 0)
    t0h = _pad_i(edge_type_nhop[:, 0], EHP, 0)
    t1h = _pad_i(edge_type_nhop[:, 1], EHP, 0)
    eenp = jnp.concatenate([edge_embed.astype(f32),
                            jnp.zeros((ENP - EN, RD), f32)], axis=0)
    rel = relation_embed.astype(f32)

    u0, u1, ns = _k1a(xpad, a0, a1, a2_0, a2_1)
    pwn = _k1b(eenp, a0, a1, a2_0, a2_1)
    or1, relpw, war2, misc = _k1c(rel, W, a0, a1, a_out, a2_0, a2_1, a2_out)

    hdrn = jnp.stack([dn, sn, tn, jnp.zeros_like(dn)]).reshape(
        4, ENP // B, B).transpose(1, 0, 2)
    hdrh = jnp.stack([dh, sh, t0h, t1h]).reshape(
        4, EHP // B, B).transpose(1, 0, 2)

    pu_cat = jnp.concatenate([ns[0], ns[1]])
    pv_cat = jnp.concatenate([ns[2], ns[3]])
    rpw_cat = jnp.concatenate([relpw[0], relpw[1]])
    acc_a = _sc_phase_a(xpad, hdrn, pwn, eenp, hdrh, rel,
                        pu_cat, pv_cat, rpw_cat)

    b0 = jnp.concatenate([a0[:, NF:2 * NF + RD],
                          jnp.zeros((NF, ROW - NF - RD), f32)], axis=1).T
    b1 = jnp.concatenate([a1[:, NF:2 * NF + RD],
                          jnp.zeros((NF, ROW - NF - RD), f32)], axis=1).T
    ad2t = a_out[:, :2 * NF].T
    h0, h1, u2, ns2 = _k2(acc_a[0], acc_a[1], u0, u1, b0, b1, ad2t, a2_out,
                          misc)

    x2cat = jnp.concatenate([h0, h1], axis=0)
    acc_b = _sc_phase_b(x2cat, hdrn, hdrh, rel, ns2[0], ns2[1], relpw[2])

    D = 2 * NF
    b2a = jnp.concatenate([a_out[:, D:D + NF].T, war2,
                           jnp.zeros((16, D), f32)], axis=0)
    b2b = jnp.concatenate([a_out[:, D + NF:2 * D].T,
                           jnp.zeros((32, D), f32)], axis=0)
    xf = _k3(acc_b[0], acc_b[1], u2, b2a, b2b)
    return xf[:N], or1


# speedup vs baseline: 1.0129x; 1.0021x over previous
import jax
import jax.numpy as jnp
from jax import lax
from jax.experimental import pallas as pl
from jax.experimental.pallas import tpu as pltpu
from jax.experimental.pallas import tpu_sc as plsc

N = 10000
NP = 10240
NF = 128
RD = 16
ROW = 160
B = 32
NTILES = 16
EN = 160000
EH = 50000
ENP = 160768
EHP = 50176
BLKN = ENP // (NTILES * B)
BLKH = EHP // (NTILES * B)
ALPHA = 0.2
NB = 4
NBLK = NP // NB


def _elu(v):
    return jnp.where(v > 0.0, v, jnp.exp(jnp.minimum(v, 0.0)) - 1.0)


def _dotT(x, w):
    return lax.dot_general(x, w, (((1,), (1,)), ((), ())),
                           preferred_element_type=jnp.float32)


def _k1_body(x_ref, ee_ref, a0_ref, a1_ref, a20_ref, a21_ref, rel_ref,
             w_ref, ao_ref, a2o_ref, u0_ref, u1_ref, ns_ref, pwn_ref,
             or1_ref, relpw_ref, war2_ref, misc_ref):
    x = x_ref[...]
    ee = ee_ref[...]
    c0 = a20_ref[...][0] @ a0_ref[...][:, 2 * NF:]
    c1 = a21_ref[...][0] @ a1_ref[...][:, 2 * NF:]
    pus, pvs = [], []
    for a_ref, a2_ref, u_ref in ((a0_ref, a20_ref, u0_ref),
                                 (a1_ref, a21_ref, u1_ref)):
        a = a_ref[...]
        a2 = a2_ref[...]
        u = _dotT(x, a[:, :NF])
        u_ref[...] = u
        pus.append(u @ a2[0])
        pvs.append(x @ (a2[0] @ a[:, NF:2 * NF]))
    z = jnp.zeros_like(pus[0])
    ns_ref[...] = jnp.stack([pus[0], pus[1], pvs[0], pvs[1], z, z, z, z])
    pwn_ref[...] = jnp.stack([ee @ c0, ee @ c1])

    @pl.when(pl.program_id(0) == 0)
    def _():
        rel = rel_ref[...]
        w = w_ref[...]
        ao = ao_ref[...]
        D = 2 * NF
        or1_ref[...] = rel @ w
        q = a2o_ref[...][0] @ ao[:, 2 * D:]
        wq = w @ q
        pad = jnp.zeros((12,), jnp.float32)
        relpw_ref[...] = jnp.stack([
            jnp.concatenate([rel @ c0, pad]),
            jnp.concatenate([rel @ c1, pad]),
            jnp.concatenate([rel @ wq, pad]),
            jnp.zeros((512,), jnp.float32)])
        war2_ref[...] = _dotT(w, ao[:, 2 * D:])
        wpv2 = a2o_ref[...][0] @ ao[:, D:2 * D]
        zz = jnp.zeros((2 * NF,), jnp.float32)
        misc_ref[...] = jnp.stack([wpv2, zz, zz, zz, zz, zz, zz, zz])


def _k2_body(acc0_ref, acc1_ref, u0_ref, u1_ref, b0_ref, b1_ref, ad2t_ref,
             a2o_ref, misc_ref, h0_ref, h1_ref, u2_ref, ns2_ref):
    ad2t = ad2t_ref[...]
    a2o = a2o_ref[...]
    wpv2 = misc_ref[...][0]
    hs = []
    for acc_ref, u_ref, b_ref, h_ref in ((acc0_ref, u0_ref, b0_ref, h0_ref),
                                         (acc1_ref, u1_ref, b1_ref, h1_ref)):
        acc = acc_ref[...]
        den = acc[:, 144]
        c = jnp.where(den == 0.0, 1e-12, den)
        rest = acc @ b_ref[...]
        h = _elu((u_ref[...] * den[:, None] + rest) / c[:, None])
        h_ref[...] = h
        hs.append(h)
    u2 = hs[0] @ ad2t[:NF] + hs[1] @ ad2t[NF:]
    u2_ref[...] = u2
    pu2 = u2 @ a2o[0]
    pv2 = hs[0] @ wpv2[:NF] + hs[1] @ wpv2[NF:]
    z = jnp.zeros_like(pu2)
    ns2_ref[...] = jnp.stack([pu2, pv2, z, z, z, z, z, z])


def _k3_body(acca_ref, accb_ref, u2_ref, b2a_ref, b2b_ref, xf_ref):
    acca = acca_ref[...]
    accb = accb_ref[...]
    den = acca[:, 144]
    c = jnp.where(den == 0.0, 1e-12, den)
    rest = acca @ b2a_ref[...] + accb @ b2b_ref[...]
    xf_ref[...] = _elu((u2_ref[...] * den[:, None] + rest) / c[:, None])


def _full(shape):
    return pl.BlockSpec(shape, lambda i: (0, 0))


_EB = ENP // NB
_k1 = pl.pallas_call(
    _k1_body, grid=(NB,),
    in_specs=[pl.BlockSpec((NBLK, NF), lambda i: (i, 0)),
              pl.BlockSpec((_EB, RD), lambda i: (i, 0)),
              _full((NF, 2 * NF + RD)), _full((NF, 2 * NF + RD)),
              _full((1, NF)), _full((1, NF)),
              _full((500, RD)), _full((RD, 2 * NF)),
              _full((2 * NF, 6 * NF)), _full((1, 2 * NF))],
    out_specs=[pl.BlockSpec((NBLK, NF), lambda i: (i, 0)),
               pl.BlockSpec((NBLK, NF), lambda i: (i, 0)),
               pl.BlockSpec((8, NBLK), lambda i: (0, i)),
               pl.BlockSpec((2, _EB), lambda i: (0, i)),
               _full((500, 2 * NF)), _full((4, 512)), _full((RD, 2 * NF)),
               _full((8, 2 * NF))],
    out_shape=[jax.ShapeDtypeStruct((NP, NF), jnp.float32),
               jax.ShapeDtypeStruct((NP, NF), jnp.float32),
               jax.ShapeDtypeStruct((8, NP), jnp.float32),
               jax.ShapeDtypeStruct((2, ENP), jnp.float32),
               jax.ShapeDtypeStruct((500, 2 * NF), jnp.float32),
               jax.ShapeDtypeStruct((4, 512), jnp.float32),
               jax.ShapeDtypeStruct((RD, 2 * NF), jnp.float32),
               jax.ShapeDtypeStruct((8, 2 * NF), jnp.float32)])

_k2 = pl.pallas_call(
    _k2_body, grid=(NB,),
    in_specs=[pl.BlockSpec((NBLK, ROW), lambda i: (i, 0)),
              pl.BlockSpec((NBLK, ROW), lambda i: (i, 0)),
              pl.BlockSpec((NBLK, NF), lambda i: (i, 0)),
              pl.BlockSpec((NBLK, NF), lambda i: (i, 0)),
              _full((ROW, NF)), _full((ROW, NF)), _full((2 * NF, 2 * NF)),
              _full((1, 2 * NF)), _full((8, 2 * NF))],
    out_specs=[pl.BlockSpec((NBLK, NF), lambda i: (i, 0)),
               pl.BlockSpec((NBLK, NF), lambda i: (i, 0)),
               pl.BlockSpec((NBLK, 2 * NF), lambda i: (i, 0)),
               pl.BlockSpec((8, NBLK), lambda i: (0, i))],
    out_shape=[jax.ShapeDtypeStruct((NP, NF), jnp.float32),
               jax.ShapeDtypeStruct((NP, NF), jnp.float32),
               jax.ShapeDtypeStruct((NP, 2 * NF), jnp.float32),
               jax.ShapeDtypeStruct((8, NP), jnp.float32)])

_k3 = pl.pallas_call(
    _k3_body, grid=(NB,),
    in_specs=[pl.BlockSpec((NBLK, ROW), lambda i: (i, 0)),
              pl.BlockSpec((NBLK, ROW), lambda i: (i, 0)),
              pl.BlockSpec((NBLK, 2 * NF), lambda i: (i, 0)),
              _full((ROW, 2 * NF)), _full((ROW, 2 * NF))],
    out_specs=pl.BlockSpec((NBLK, 2 * NF), lambda i: (i, 0)),
    out_shape=jax.ShapeDtypeStruct((NP, 2 * NF), jnp.float32))


_KEYS = ("hv", "dv", "sv", "dvo", "svo", "t0v", "t1v", "dsc",
         "pwv", "pug", "pvg", "eev", "r1v", "xrows", "orows")


def _set_types():
    return [
        pltpu.VMEM((4, B), jnp.int32),
        pltpu.VMEM((B,), jnp.int32),
        pltpu.VMEM((B,), jnp.int32),
        pltpu.VMEM((B,), jnp.int32),
        pltpu.VMEM((B,), jnp.int32),
        pltpu.VMEM((B,), jnp.int32),
        pltpu.VMEM((B,), jnp.int32),
        pltpu.VMEM((B,), jnp.int32),
        pltpu.VMEM((B,), jnp.float32),
        pltpu.VMEM((B,), jnp.float32),
        pltpu.VMEM((B,), jnp.float32),
        pltpu.VMEM((B, RD), jnp.float32),
        pltpu.VMEM((B, RD), jnp.float32),
        pltpu.VMEM((B, NF), jnp.float32),
        pltpu.VMEM((B, ROW), jnp.float32),
    ]


def _make_sc_edge_pass(phase_b):
    mesh = plsc.VectorSubcoreMesh(core_axis_name="c", subcore_axis_name="s")
    nset = len(_KEYS)
    scratch = _set_types() + _set_types() + [
        pltpu.VMEM((B,), jnp.float32),
        pltpu.VMEM((512,), jnp.float32),
        pltpu.SemaphoreType.DMA,
        pltpu.SemaphoreType.DMA,
        pltpu.SemaphoreType.DMA,
        pltpu.SemaphoreType.DMA,
        pltpu.SemaphoreType.DMA,
        pltpu.VMEM_SHARED((NP, ROW), jnp.float32),
    ]

    def body(*refs):
        if phase_b:
            (x_hbm, hdrn, hdrh, rel_hbm, pu_t, pv_t, rpw_t, out_hbm) = refs[:8]
            nin = 8
        else:
            (x_hbm, hdrn, pwn, een, hdrh, rel_hbm, pu_t, pv_t, rpw_t,
             out_hbm) = refs[:10]
            nin = 10
        sets = [dict(zip(_KEYS, refs[nin:nin + nset])),
                dict(zip(_KEYS, refs[nin + nset:nin + 2 * nset]))]
        (ebuf, rpw_vm, sem_l, sem_g0, sem_g1, sem_s0, sem_s1,
         acc) = refs[nin + 2 * nset:]
        sem_g = (sem_g0, sem_g1)
        sem_s = (sem_s0, sem_s1)
        cid = lax.axis_index("c")
        sid = lax.axis_index("s")
        node_off = cid * NP
        lane = lax.broadcasted_iota(jnp.int32, (16,), 0)
        if phase_b:
            pltpu.sync_copy(rpw_t, rpw_vm)
        else:
            pltpu.sync_copy(rpw_t.at[pl.ds(cid * 512, 512)], rpw_vm)

        z0 = sets[0]["orows"]
        zero = jnp.zeros((16,), jnp.float32)

        def zrow(i, carry):
            for st in sets:
                for j in range(ROW // 16):
                    st["orows"][i, pl.ds(j * 16, 16)] = zero
            return carry

        lax.fori_loop(0, B, zrow, 0)
        r0 = sid * (NP // NTILES)
        zcps = [pltpu.async_copy(z0, acc.at[pl.ds(r0 + z * B, B)], sem_l)
                for z in range((NP // NTILES) // B)]
        for cp in zcps:
            cp.wait()
        plsc.subcore_barrier()

        zi = jnp.zeros((16,), jnp.int32)
        for st, sem in ((sets[0], sem_s0), (sets[1], sem_s1)):
            for j in range(B // 16):
                st["dsc"][pl.ds(j * 16, 16)] = zi
            pltpu.async_copy(st["orows"], acc.at[st["dsc"]], sem, add=True)

        def hdr_tab(kind):
            return hdrh if kind else hdrn

        def fire_lin(kind, blk, s):
            pltpu.async_copy(hdr_tab(kind).at[blk], s["hv"], sem_l)
            if kind == 0 and not phase_b:
                base = blk * B
                pltpu.async_copy(pwn.at[cid, pl.ds(base, B)], s["pwv"], sem_l)
                pltpu.async_copy(een.at[pl.ds(base, B)], s["eev"], sem_l)

        def wait_lin(kind, s):
            pltpu.make_async_copy(hdr_tab(kind).at[0], s["hv"], sem_l).wait()
            if kind == 0 and not phase_b:
                pltpu.make_async_copy(pwn.at[0, pl.ds(0, B)], s["pwv"],
                                      sem_l).wait()
                pltpu.make_async_copy(een.at[pl.ds(0, B)], s["eev"],
                                      sem_l).wait()

        def unpack(kind, s):
            rows = 4 if kind else (3 if phase_b else 2)
            obn = jnp.broadcast_to(node_off, (16,)).astype(jnp.int32)
            hv = s["hv"]
            for j in range(B // 16):
                sl = pl.ds(j * 16, 16)
                d16 = hv[0, sl]
                s16 = hv[1, sl]
                s["dv"][sl] = d16
                s["sv"][sl] = s16
                s["svo"][sl] = s16 + obn
                if not phase_b:
                    s["dvo"][sl] = d16 + obn
                if rows >= 3:
                    s["t0v"][sl] = hv[2, sl]
                if rows >= 4:
                    s["t1v"][sl] = hv[3, sl]

        def gath_list(kind, s):
            if kind == 0:
                if phase_b:
                    return [(rel_hbm.at[s["t0v"]], s["eev"]),
                            (pu_t.at[s["dv"]], s["pug"]),
                            (pv_t.at[s["sv"]], s["pvg"]),
                            (x_hbm.at[s["svo"]], s["xrows"])]
                return [(pu_t.at[s["dvo"]], s["pug"]),
                        (pv_t.at[s["svo"]], s["pvg"]),
                        (x_hbm.at[s["sv"]], s["xrows"])]
            return [(rel_hbm.at[s["t0v"]], s["eev"]),
                    (rel_hbm.at[s["t1v"]], s["r1v"]),
                    (pu_t.at[s["dv"] if phase_b else s["dvo"]], s["pug"]),
                    (pv_t.at[s["sv"] if phase_b else s["svo"]], s["pvg"]),
                    (x_hbm.at[s["svo"] if phase_b else s["sv"]], s["xrows"])]

        def fire_gath(kind, p):
            for src, dst in gath_list(kind, sets[p]):
                pltpu.async_copy(src, dst, sem_g[p])

        def wait_gath(kind, p):
            for src, dst in gath_list(kind, sets[p]):
                pltpu.make_async_copy(src, dst, sem_g[p]).wait()

        def compute_e(kind, s):
            for j in range(B // 16):
                sl = pl.ds(j * 16, 16)
                p = s["pug"][sl] + s["pvg"][sl]
                if kind == 0:
                    if phase_b:
                        p = p + plsc.load_gather(rpw_vm, [s["t0v"][sl]])
                    else:
                        p = p + s["pwv"][sl]
                else:
                    p = p + plsc.load_gather(rpw_vm, [s["t0v"][sl]])
                    p = p + plsc.load_gather(rpw_vm, [s["t1v"][sl]])
                ebuf[sl] = jnp.exp(jnp.minimum(-p, -ALPHA * p))

        def drain_scatter(p):
            s = sets[p]
            pltpu.make_async_copy(s["orows"], acc.at[s["dsc"]],
                                  sem_s[p]).wait()

        def scale_fire_scatter(kind, p):
            s = sets[p]
            orows, xrows, eev, r1v = (s["orows"], s["xrows"], s["eev"],
                                      s["r1v"])

            def sblk(j, carry):
                ev16 = ebuf[pl.ds(j * 16, 16)]
                for k in range(16):
                    i = j * 16 + k
                    ev = jnp.broadcast_to(ev16[k], (16,))
                    for c in range(NF // 16):
                        sl = pl.ds(c * 16, 16)
                        orows[i, sl] = xrows[i, sl] * ev
                    ee_row = eev[i, :]
                    if kind:
                        ee_row = ee_row + r1v[i, :]
                    orows[i, pl.ds(NF, 16)] = ee_row * ev
                    orows[i, pl.ds(NF + 16, 16)] = jnp.where(lane == 0, ev,
                                                             0.0)
                return carry

            lax.fori_loop(0, B // 16, sblk, 0)
            for j in range(B // 16):
                sl = pl.ds(j * 16, 16)
                s["dsc"][sl] = s["dv"][sl]
            pltpu.async_copy(orows, acc.at[s["dsc"]], sem_s[p], add=True)

        def run_kind(kind, nblk):
            blk0 = sid * nblk
            fire_lin(kind, blk0, sets[0])
            wait_lin(kind, sets[0])
            unpack(kind, sets[0])
            fire_gath(kind, 0)
            fire_lin(kind, blk0 + 1, sets[1])
            npair = nblk // 2

            def half(k, p, b_next2):
                q = 1 - p
                wait_lin(kind, sets[q])
                unpack(kind, sets[q])
                fire_gath(kind, q)
                wait_gath(kind, p)
                compute_e(kind, sets[p])
                drain_scatter(p)
                scale_fire_scatter(kind, p)
                fire_lin(kind, b_next2, sets[p])

            def pair(k, carry):
                b0 = blk0 + 2 * k
                half(k, 0, b0 + 2)
                half(k, 1, b0 + 3)
                return carry

            lax.fori_loop(0, npair - 1, pair, 0)
            wait_lin(kind, sets[1])
            unpack(kind, sets[1])
            fire_gath(kind, 1)
            wait_gath(kind, 0)
            compute_e(kind, sets[0])
            drain_scatter(0)
            scale_fire_scatter(kind, 0)
            wait_gath(kind, 1)
            compute_e(kind, sets[1])
            drain_scatter(1)
            scale_fire_scatter(kind, 1)

        run_kind(0, BLKN)
        run_kind(1, BLKH)
        drain_scatter(0)
        drain_scatter(1)
        plsc.subcore_barrier()
        pltpu.sync_copy(acc.at[pl.ds(r0, NP // NTILES)],
                        out_hbm.at[cid, pl.ds(r0, NP // NTILES)])

    return pl.kernel(body,
                     out_type=jax.ShapeDtypeStruct((2, NP, ROW), jnp.float32),
                     mesh=mesh, scratch_types=scratch,
                     compiler_params=pltpu.CompilerParams(
                         needs_layout_passes=False,
                         use_tc_tiling_on_sc=False))


_sc_phase_a = _make_sc_edge_pass(phase_b=False)
_sc_phase_b = _make_sc_edge_pass(phase_b=True)


def _pad_i(a, L, fill):
    a = a.astype(jnp.int32)
    return jnp.concatenate([a, jnp.full((L - a.shape[0],), fill, jnp.int32)])


def kernel(Corpus_, batch_inputs, entity_embeddings, relation_embed,
           edge_list, edge_type, edge_embed, edge_list_nhop, edge_type_nhop,
           a0, a2_0, a1, a2_1, W, a_out, a2_out):
    f32 = jnp.float32
    x = entity_embeddings.astype(f32)
    xpad = jnp.concatenate([x, jnp.zeros((NP - N, NF), f32)], axis=0)
    dn = _pad_i(edge_list[0], ENP, N)
    sn = _pad_i(edge_list[1], ENP, 0)
    tn = _pad_i(edge_type, ENP, 0)
    dh = _pad_i(edge_list_nhop[0], EHP, N)
    sh = _pad_i(edge_list_nhop[1], EHP, 0)
    t0h = _pad_i(edge_type_nhop[:, 0], EHP, 0)
    t1h = _pad_i(edge_type_nhop[:, 1], EHP, 0)
    eenp = jnp.concatenate([edge_embed.astype(f32),
                            jnp.zeros((ENP - EN, RD), f32)], axis=0)
    rel = relation_embed.astype(f32)

    u0, u1, ns, pwn, or1, relpw, war2, misc = _k1(
        xpad, eenp, a0, a1, a2_0, a2_1, rel, W, a_out, a2_out)

    hdrn = jnp.stack([dn, sn, tn, jnp.zeros_like(dn)]).reshape(
        4, ENP // B, B).transpose(1, 0, 2)
    hdrh = jnp.stack([dh, sh, t0h, t1h]).reshape(
        4, EHP // B, B).transpose(1, 0, 2)

    pu_cat = jnp.concatenate([ns[0], ns[1]])
    pv_cat = jnp.concatenate([ns[2], ns[3]])
    rpw_cat = jnp.concatenate([relpw[0], relpw[1]])
    acc_a = _sc_phase_a(xpad, hdrn, pwn, eenp, hdrh, rel,
                        pu_cat, pv_cat, rpw_cat)

    b0 = jnp.concatenate([a0[:, NF:2 * NF + RD],
                          jnp.zeros((NF, ROW - NF - RD), f32)], axis=1).T
    b1 = jnp.concatenate([a1[:, NF:2 * NF + RD],
                          jnp.zeros((NF, ROW - NF - RD), f32)], axis=1).T
    ad2t = a_out[:, :2 * NF].T
    h0, h1, u2, ns2 = _k2(acc_a[0], acc_a[1], u0, u1, b0, b1, ad2t, a2_out,
                          misc)

    x2cat = jnp.concatenate([h0, h1], axis=0)
    acc_b = _sc_phase_b(x2cat, hdrn, hdrh, rel, ns2[0], ns2[1], relpw[2])

    D = 2 * NF
    b2a = jnp.concatenate([a_out[:, D:D + NF].T, war2,
                           jnp.zeros((16, D), f32)], axis=0)
    b2b = jnp.concatenate([a_out[:, D + NF:2 * D].T,
                           jnp.zeros((32, D), f32)], axis=0)
    xf = _k3(acc_b[0], acc_b[1], u2, b2a, b2b)
    return xf[:N], or1
